# baseline (speedup 1.0000x reference)
import functools

import jax
import jax.numpy as jnp
from jax import lax
from jax.experimental import pallas as pl
from jax.experimental.pallas import tpu as pltpu
from jax.experimental.pallas import tpu_sc as plsc

D_MODEL = 64
SCALE = 8.0

NC = 2
NS = 16
NW = NC * NS
LANES = 16

CHUNK = 128
NB = 4


def _sc_embed(x2d, table, n_rows):
    rows_per_w = n_rows // NW
    assert rows_per_w % NB == 0
    b_total = n_rows * CHUNK
    mesh = plsc.VectorSubcoreMesh(core_axis_name="c", subcore_axis_name="s")

    scratch = (
        [pltpu.VMEM((rows_per_w, CHUNK), jnp.int32)]
        + [pltpu.VMEM((CHUNK, D_MODEL), jnp.float32) for _ in range(2 * NB)]
        + [pltpu.SemaphoreType.DMA for _ in range(2 * NB)]
    )

    @functools.partial(
        pl.kernel,
        out_type=jax.ShapeDtypeStruct((b_total, D_MODEL), jnp.float32),
        mesh=mesh,
        scratch_types=scratch,
        compiler_params=pltpu.CompilerParams(use_tc_tiling_on_sc=False),
    )
    def k(x_hbm, table_hbm, out_hbm, idx_v, *bufs_and_sems):
        inb = bufs_and_sems[:NB]
        outb = bufs_and_sems[NB:2 * NB]
        sem_in = bufs_and_sems[2 * NB:3 * NB]
        sem_out = bufs_and_sems[3 * NB:4 * NB]

        wid = lax.axis_index("s") * NC + lax.axis_index("c")
        base_row = wid * rows_per_w
        pltpu.sync_copy(x_hbm.at[pl.ds(base_row, rows_per_w)], idx_v)

        for b in range(NB):
            pltpu.async_copy(table_hbm.at[idx_v.at[b]], inb[b], sem_in[b])

        @pl.loop(0, rows_per_w, step=NB)
        def _outer(i):
            for b in range(NB):
                c = i + b
                pltpu.make_async_copy(
                    table_hbm.at[idx_v.at[c]], inb[b], sem_in[b]).wait()

                @pl.when(c >= NB)
                def _():
                    pltpu.make_async_copy(
                        outb[b], out_hbm.at[pl.ds(0, CHUNK)], sem_out[b]).wait()

                @pl.loop(0, CHUNK, unroll=8)
                def _row(r):
                    for d in range(D_MODEL // LANES):
                        sl = pl.ds(d * LANES, LANES)
                        outb[b][r, sl] = inb[b][r, sl] * SCALE

                @pl.when(c + NB < rows_per_w)
                def _():
                    pltpu.async_copy(
                        table_hbm.at[idx_v.at[c + NB]], inb[b], sem_in[b])

                out_base = (base_row + c) * CHUNK
                pltpu.async_copy(
                    outb[b], out_hbm.at[pl.ds(out_base, CHUNK)], sem_out[b])

        for b in range(NB):
            pltpu.make_async_copy(
                outb[b], out_hbm.at[pl.ds(0, CHUNK)], sem_out[b]).wait()

    return k(x2d, table)


def kernel(x, table):
    b, s = x.shape
    total = b * s
    n_rows = total // CHUNK
    x2d = x.reshape(n_rows, CHUNK).astype(jnp.int32)
    out = _sc_embed(x2d, table, n_rows)
    return out.reshape(b, s, D_MODEL)
